# baseline (speedup 1.0000x reference)
import jax
import jax.numpy as jnp
import numpy as np
from jax.experimental import pallas as pl

_TEMP = np.float32(0.7)


def _body(h_ref, w_ref, o_ref):
    o_ref[...] = jnp.dot(
        h_ref[...], w_ref[...], preferred_element_type=jnp.float32
    ) / _TEMP


def kernel(hidden_states, gate_weight):
    B, S, D = hidden_states.shape
    E = gate_weight.shape[0]
    T = B * S
    h = hidden_states.reshape(T, D)
    wT = gate_weight.T

    TB = 512
    out = pl.pallas_call(
        _body,
        grid=(T // TB,),
        in_specs=[
            pl.BlockSpec((TB, D), lambda i: (i, 0)),
            pl.BlockSpec((D, E), lambda i: (0, 0)),
        ],
        out_specs=pl.BlockSpec((TB, E), lambda i: (i, 0)),
        out_shape=jax.ShapeDtypeStruct((T, E), jnp.float32),
    )(h, wT)
    return out.reshape(B, S, E)


# speedup vs baseline: 1.1092x; 1.1092x over previous
import jax
import jax.numpy as jnp
import numpy as np
from jax.experimental import pallas as pl

_TEMP = np.float32(0.7)


def _body(h_ref, w_ref, o_ref):
    o_ref[...] = jnp.dot(
        h_ref[...], w_ref[...], preferred_element_type=jnp.float32
    ) / _TEMP


def kernel(hidden_states, gate_weight):
    B, S, D = hidden_states.shape
    E = gate_weight.shape[0]
    T = B * S
    h = hidden_states.reshape(T, D)
    wT = gate_weight.T

    TB = 1024
    out = pl.pallas_call(
        _body,
        grid=(T // TB,),
        in_specs=[
            pl.BlockSpec((TB, D), lambda i: (i, 0)),
            pl.BlockSpec((D, E), lambda i: (0, 0)),
        ],
        out_specs=pl.BlockSpec((TB, E), lambda i: (i, 0)),
        out_shape=jax.ShapeDtypeStruct((T, E), jnp.float32),
    )(h, wT)
    return out.reshape(B, S, E)
